# baseline (speedup 1.0000x reference)
import functools

import jax
import jax.numpy as jnp
from jax import lax
from jax.experimental import pallas as pl
from jax.experimental.pallas import tpu as pltpu
from jax.experimental.pallas import tpu_sc as plsc

_LANES = 16
_CHUNK = 32


def kernel(inputs, pos_table):
    batch, seq, dim = inputs.shape
    info = plsc.get_sparse_core_info()
    nw = info.num_cores * info.num_subcores
    seq_per_w = seq // nw
    n_chunks = seq_per_w // _CHUNK
    mesh = plsc.VectorSubcoreMesh(core_axis_name="c", subcore_axis_name="s")

    @functools.partial(
        pl.kernel,
        mesh=mesh,
        out_type=jax.ShapeDtypeStruct((batch, seq, dim), jnp.float32),
        scratch_types=[
            pltpu.VMEM((_CHUNK, dim), jnp.float32),
            pltpu.VMEM((_CHUNK, dim), jnp.float32),
        ],
    )
    def sc_kernel(in_hbm, pos_hbm, out_hbm, pos_v, buf_v):
        wid = lax.axis_index("s") * info.num_cores + lax.axis_index("c")
        seq0 = wid * seq_per_w

        def chunk_body(ci, _):
            row0 = seq0 + ci * _CHUNK
            pltpu.sync_copy(pos_hbm.at[pl.ds(row0, _CHUNK), :], pos_v)

            def batch_body(b, _):
                pltpu.sync_copy(in_hbm.at[b, pl.ds(row0, _CHUNK), :], buf_v)

                def row_body(r, _):
                    def vec_body(v, _):
                        sl = pl.ds(v * _LANES, _LANES)
                        buf_v[r, sl] = buf_v[r, sl] + pos_v[r, sl]
                        return 0

                    return lax.fori_loop(0, dim // _LANES, vec_body, 0)

                lax.fori_loop(0, _CHUNK, row_body, 0)
                pltpu.sync_copy(buf_v, out_hbm.at[b, pl.ds(row0, _CHUNK), :])
                return 0

            lax.fori_loop(0, batch, batch_body, 0)
            return 0

        lax.fori_loop(0, n_chunks, chunk_body, 0)

    return sc_kernel(inputs, pos_table)


# speedup vs baseline: 2.0963x; 2.0963x over previous
import functools

import jax
import jax.numpy as jnp
from jax import lax
from jax.experimental import pallas as pl
from jax.experimental.pallas import tpu as pltpu
from jax.experimental.pallas import tpu_sc as plsc

_LANES = 16
_CHUNK = 32


def kernel(inputs, pos_table):
    batch, seq, dim = inputs.shape
    info = plsc.get_sparse_core_info()
    nw = info.num_cores * info.num_subcores
    seq_per_w = seq // nw
    n_chunks = seq_per_w // _CHUNK
    n_steps = n_chunks * batch
    mesh = plsc.VectorSubcoreMesh(core_axis_name="c", subcore_axis_name="s")

    @functools.partial(
        pl.kernel,
        mesh=mesh,
        out_type=jax.ShapeDtypeStruct((batch, seq, dim), jnp.float32),
        scratch_types=[
            pltpu.VMEM((_CHUNK, dim), jnp.float32),
            pltpu.VMEM((_CHUNK, dim), jnp.float32),
            pltpu.VMEM((_CHUNK, dim), jnp.float32),
            pltpu.SemaphoreType.DMA,
            pltpu.SemaphoreType.DMA,
            pltpu.SemaphoreType.DMA,
            pltpu.SemaphoreType.DMA,
            pltpu.SemaphoreType.DMA,
        ],
    )
    def sc_kernel(in_hbm, pos_hbm, out_hbm, buf0, buf1, pos_v,
                  s_in0, s_in1, s_out0, s_out1, s_pos):
        wid = lax.axis_index("s") * info.num_cores + lax.axis_index("c")
        seq0 = wid * seq_per_w
        bufs = (buf0, buf1)
        in_sems = (s_in0, s_in1)
        out_sems = (s_out0, s_out1)
        steps = [(ci, b) for ci in range(n_chunks) for b in range(batch)]

        pos_h = pltpu.async_copy(
            pos_hbm.at[pl.ds(seq0, _CHUNK), :], pos_v, s_pos)
        in_h = {0: pltpu.async_copy(
            in_hbm.at[0, pl.ds(seq0, _CHUNK), :], bufs[0], in_sems[0])}
        out_h = {}

        for s, (ci, b) in enumerate(steps):
            row0 = seq0 + ci * _CHUNK
            p = s % 2
            if s + 1 < n_steps:
                ci2, b2 = steps[s + 1]
                if s - 1 >= 0:
                    out_h[s - 1].wait()
                in_h[s + 1] = pltpu.async_copy(
                    in_hbm.at[b2, pl.ds(seq0 + ci2 * _CHUNK, _CHUNK), :],
                    bufs[(s + 1) % 2], in_sems[(s + 1) % 2])
            if b == 0:
                pos_h.wait()
            in_h[s].wait()

            buf = bufs[p]

            def row_body(r, _, buf=buf):
                for v in range(dim // _LANES):
                    sl = pl.ds(v * _LANES, _LANES)
                    buf[r, sl] = buf[r, sl] + pos_v[r, sl]
                return 0

            lax.fori_loop(0, _CHUNK, row_body, 0)

            if b == batch - 1 and ci + 1 < n_chunks:
                pos_h = pltpu.async_copy(
                    pos_hbm.at[pl.ds(seq0 + (ci + 1) * _CHUNK, _CHUNK), :],
                    pos_v, s_pos)
            out_h[s] = pltpu.async_copy(
                buf, out_hbm.at[b, pl.ds(row0, _CHUNK), :], out_sems[p])

        out_h[n_steps - 2].wait()
        out_h[n_steps - 1].wait()

    return sc_kernel(inputs, pos_table)
